# Initial kernel scaffold; baseline (speedup 1.0000x reference)
import jax
import jax.numpy as jnp
from jax.experimental import pallas as pl


def kernel(delta):
    raise NotImplementedError("write your pallas kernel here")


# speedup vs baseline: 19.7068x; 19.7068x over previous
import functools

import jax
import jax.numpy as jnp
from jax.experimental import pallas as pl

_SIGNIF_ARG = 0.1


def _threshold_kernel(k, delta_ref, thr_ref):
    d = delta_ref[...]
    valid = jnp.isfinite(d)
    bits = jax.lax.bitcast_convert_type(jnp.abs(d), jnp.int32)
    bits = jnp.where(valid, bits, -1)

    def body(i, t):
        b = 30 - i
        cand = t | jax.lax.shift_left(jnp.int32(1), b)
        cnt = jnp.sum((bits >= cand).astype(jnp.int32), axis=1, keepdims=True)
        return jnp.where(cnt >= k, cand, t)

    t0 = jnp.zeros((d.shape[0], 1), jnp.int32)
    thr_ref[...] = jax.lax.fori_loop(0, 31, body, t0)


def _mask_kernel(delta_ref, thr_ref, sig_ref, dir_ref, rank_ref, valid_ref):
    d = delta_ref[...]
    thr = thr_ref[...]
    absd = jnp.abs(d)
    valid = jnp.isfinite(d)
    bits = jax.lax.bitcast_convert_type(absd, jnp.int32)
    bits = jnp.where(valid, bits, -1)
    sig_ref[...] = (bits >= thr) & valid
    dir_ref[...] = (d > 0).astype(jnp.int32)
    rank_ref[...] = absd
    valid_ref[...] = valid


def kernel(delta):
    B, N = delta.shape
    k = max(1, int(N * _SIGNIF_ARG))

    thr = pl.pallas_call(
        functools.partial(_threshold_kernel, k),
        out_shape=jax.ShapeDtypeStruct((B, 1), jnp.int32),
    )(delta)

    blk = 4096
    nblk = N // blk
    out_shape = [
        jax.ShapeDtypeStruct((B, N), jnp.bool_),
        jax.ShapeDtypeStruct((B, N), jnp.int32),
        jax.ShapeDtypeStruct((B, N), jnp.float32),
        jax.ShapeDtypeStruct((B, N), jnp.bool_),
    ]
    out_specs = [pl.BlockSpec((B, blk), lambda j: (0, j)) for _ in range(4)]
    de_sig, de_dir, rank_score, valid_mask = pl.pallas_call(
        _mask_kernel,
        grid=(nblk,),
        in_specs=[
            pl.BlockSpec((B, blk), lambda j: (0, j)),
            pl.BlockSpec((B, 1), lambda j: (0, 0)),
        ],
        out_specs=out_specs,
        out_shape=out_shape,
    )(delta, thr)
    return (de_sig, de_dir.astype(jnp.int64), rank_score, valid_mask)
